# baseline (speedup 1.0000x reference)
import jax
import jax.numpy as jnp
from jax.experimental import pallas as pl
from jax.experimental.pallas import tpu as pltpu

_NROWS = 26
_SUB = 16384
_SPLIT = 8
_BLK = _SUB // _SPLIT


def _copy_body(map_ref, in_ref, out_ref):
    out_ref[...] = in_ref[...]


def kernel(mamdani_output, mapping):
    idx = mapping.reshape(_NROWS).astype(jnp.int32)
    out = pl.pallas_call(
        _copy_body,
        grid_spec=pltpu.PrefetchScalarGridSpec(
            num_scalar_prefetch=1,
            grid=(_NROWS, _SPLIT),
            in_specs=[pl.BlockSpec((1, _BLK, 32), lambda i, j, m: (m[i], j, 0))],
            out_specs=pl.BlockSpec((1, _BLK, 32), lambda i, j, m: (i, j, 0)),
        ),
        out_shape=jax.ShapeDtypeStruct((_NROWS, _SUB, 32), jnp.float32),
    )(idx, mamdani_output)
    return jnp.expand_dims(out, 1)


# speedup vs baseline: 1.1149x; 1.1149x over previous
import jax
import jax.numpy as jnp
from jax.experimental import pallas as pl
from jax.experimental.pallas import tpu as pltpu

_NROWS = 26
_SUB = 16384
_SPLIT = 1
_BLK = _SUB // _SPLIT


def _copy_body(map_ref, in_ref, out_ref):
    out_ref[...] = in_ref[...]


def kernel(mamdani_output, mapping):
    idx = mapping.reshape(_NROWS).astype(jnp.int32)
    out = pl.pallas_call(
        _copy_body,
        grid_spec=pltpu.PrefetchScalarGridSpec(
            num_scalar_prefetch=1,
            grid=(_NROWS, _SPLIT),
            in_specs=[pl.BlockSpec((1, _BLK, 32), lambda i, j, m: (m[i], j, 0))],
            out_specs=pl.BlockSpec((1, _BLK, 32), lambda i, j, m: (i, j, 0)),
        ),
        out_shape=jax.ShapeDtypeStruct((_NROWS, _SUB, 32), jnp.float32),
    )(idx, mamdani_output)
    return jnp.expand_dims(out, 1)


# speedup vs baseline: 5.9274x; 5.3165x over previous
import jax
import jax.numpy as jnp
from jax.experimental import pallas as pl
from jax.experimental.pallas import tpu as pltpu

_NROWS = 26
_MINOR = 16384
_SPLIT = 8
_BLK = _MINOR // _SPLIT


def _copy_body(map_ref, in_ref, out_ref):
    out_ref[...] = in_ref[...]


def kernel(mamdani_output, mapping):
    src = jnp.transpose(mamdani_output, (0, 2, 1))
    idx = mapping.reshape(_NROWS)
    out = pl.pallas_call(
        _copy_body,
        grid_spec=pltpu.PrefetchScalarGridSpec(
            num_scalar_prefetch=1,
            grid=(_NROWS, _SPLIT),
            in_specs=[pl.BlockSpec((1, 32, _BLK), lambda i, j, m: (m[i], 0, j))],
            out_specs=pl.BlockSpec((1, 32, _BLK), lambda i, j, m: (i, 0, j)),
        ),
        out_shape=jax.ShapeDtypeStruct((_NROWS, 32, _MINOR), jnp.float32),
    )(idx, src)
    return jnp.expand_dims(jnp.transpose(out, (0, 2, 1)), 1)


# speedup vs baseline: 19.3627x; 3.2667x over previous
import jax
import jax.numpy as jnp
from jax.experimental import pallas as pl
from jax.experimental.pallas import tpu as pltpu

_NROWS = 26
_MINOR = 16384
_SPLIT = 1
_BLK = _MINOR // _SPLIT


def _copy_body(map_ref, in_ref, out_ref):
    out_ref[...] = in_ref[...]


def kernel(mamdani_output, mapping):
    src = jnp.transpose(mamdani_output, (0, 2, 1))
    idx = mapping.reshape(_NROWS)
    out = pl.pallas_call(
        _copy_body,
        grid_spec=pltpu.PrefetchScalarGridSpec(
            num_scalar_prefetch=1,
            grid=(_NROWS, _SPLIT),
            in_specs=[pl.BlockSpec((1, 32, _BLK), lambda i, j, m: (m[i], 0, j))],
            out_specs=pl.BlockSpec((1, 32, _BLK), lambda i, j, m: (i, 0, j)),
        ),
        out_shape=jax.ShapeDtypeStruct((_NROWS, 32, _MINOR), jnp.float32),
    )(idx, src)
    return jnp.expand_dims(jnp.transpose(out, (0, 2, 1)), 1)
